# Initial kernel scaffold; baseline (speedup 1.0000x reference)
import jax
import jax.numpy as jnp
from jax.experimental import pallas as pl


def kernel(x, edge_index, W1, b1, W2, b2, W_fc, b_fc):
    raise NotImplementedError("write your pallas kernel here")


# speedup vs baseline: 6.3954x; 6.3954x over previous
import functools

import jax
import jax.numpy as jnp
from jax import lax
from jax.experimental import pallas as pl
from jax.experimental.pallas import tpu as pltpu
from jax.experimental.pallas import tpu_sc as plsc

N, E, DIN, H = 10000, 320000, 128, 246

NC, NS, LANES = 2, 16, 16
NPAD = 10240
F = 256
FH = F // NC
CHUNK = 128
EC = 160
EPAD = NS * EC * CHUNK
ECH = EC // NC
BCH = 16
RPT = NPAD // NS
RB = 640
NBLK = NPAD // RB


def _deg_body(srcg, dstg, ones_h, zeros_h, out, sidx, didx, ones_v,
              od_sh, id_sh):
    c = lax.axis_index("c")
    s = lax.axis_index("s")
    pltpu.sync_copy(zeros_h, od_sh.at[pl.ds(s * RPT, RPT)])
    pltpu.sync_copy(zeros_h, id_sh.at[pl.ds(s * RPT, RPT)])
    pltpu.sync_copy(ones_h, ones_v)
    pltpu.sync_copy(srcg.at[0, s, pl.ds(c * ECH, ECH)], sidx)
    pltpu.sync_copy(dstg.at[s, pl.ds(c * ECH, ECH)], didx)
    plsc.subcore_barrier()

    def body(j, carry):
        pltpu.sync_copy(ones_v, od_sh.at[sidx.at[j]], add=True)
        pltpu.sync_copy(ones_v, id_sh.at[didx.at[j]], add=True)
        return carry

    lax.fori_loop(0, ECH, body, 0)
    plsc.subcore_barrier()
    pltpu.sync_copy(od_sh.at[pl.ds(s * RPT, RPT)],
                    out.at[c, 0, pl.ds(s * RPT, RPT)])
    pltpu.sync_copy(id_sh.at[pl.ds(s * RPT, RPT)],
                    out.at[c, 1, pl.ds(s * RPT, RPT)])


@functools.lru_cache(maxsize=None)
def _sc_calls():
    mesh = plsc.VectorSubcoreMesh(core_axis_name="c", subcore_axis_name="s",
                                  num_cores=NC, num_subcores=NS)
    deg_call = pl.kernel(
        _deg_body,
        out_type=jax.ShapeDtypeStruct((NC, 2, NPAD), jnp.float32),
        mesh=mesh,
        scratch_types=[
            pltpu.VMEM((ECH, CHUNK), jnp.int32),
            pltpu.VMEM((ECH, CHUNK), jnp.int32),
            pltpu.VMEM((CHUNK,), jnp.float32),
            pltpu.VMEM_SHARED((NPAD,), jnp.float32),
            pltpu.VMEM_SHARED((NPAD,), jnp.float32),
        ],
    )
    seg_call = pl.kernel(
        _seg_body,
        out_type=jax.ShapeDtypeStruct((NC, NPAD, FH), jnp.float32),
        mesh=mesh,
        scratch_types=[
            pltpu.VMEM((BCH, CHUNK), jnp.int32),
            pltpu.VMEM((BCH, CHUNK), jnp.int32),
            pltpu.VMEM((CHUNK, FH), jnp.float32),
            pltpu.VMEM_SHARED((NPAD, FH), jnp.float32),
            pltpu.SemaphoreType.DMA,
        ],
    )
    return deg_call, seg_call


def _seg_body(srcg, dstg, table, zeros_h, out, sidx, didx, rows_v, acc_sh,
              sem):
    c = lax.axis_index("c")
    s = lax.axis_index("s")
    pltpu.sync_copy(zeros_h, acc_sh.at[pl.ds(s * RPT, RPT)])
    plsc.subcore_barrier()

    def blk(b, carry):
        pltpu.sync_copy(srcg.at[c, s, pl.ds(b * BCH, BCH)], sidx)
        pltpu.sync_copy(dstg.at[s, pl.ds(b * BCH, BCH)], didx)

        def body(j, carry2):
            pltpu.async_copy(table.at[sidx.at[j]], rows_v, sem).wait()
            pltpu.sync_copy(rows_v, acc_sh.at[didx.at[j]], add=True)
            return carry2

        lax.fori_loop(0, BCH, body, 0)
        return carry

    lax.fori_loop(0, EC // BCH, blk, 0)
    plsc.subcore_barrier()
    pltpu.sync_copy(acc_sh.at[pl.ds(s * RPT, RPT)],
                    out.at[c, pl.ds(s * RPT, RPT)])


def _mm1_body(x_ref, od_ref, w_ref, out_ref):
    od = od_ref[0] + od_ref[1]
    nsrc = jnp.where(od > 0, lax.rsqrt(jnp.maximum(od, 1.0)), 0.0)
    y = jnp.dot(x_ref[...] * nsrc[:, None], w_ref[...],
                preferred_element_type=jnp.float32)
    out_ref[0] = y[:, :FH]
    out_ref[1] = y[:, FH:]


def _mm1(xp, odp, w1p):
    return pl.pallas_call(
        _mm1_body,
        grid=(NBLK,),
        in_specs=[
            pl.BlockSpec((RB, DIN), lambda i: (i, 0)),
            pl.BlockSpec((NC, RB), lambda i: (0, i)),
            pl.BlockSpec((DIN, F), lambda i: (0, 0)),
        ],
        out_specs=pl.BlockSpec((NC, RB, FH), lambda i: (0, i, 0)),
        out_shape=jax.ShapeDtypeStruct((NC, NPAD, FH), jnp.float32),
    )(xp, odp, w1p)


def _mm2_body(agg_ref, od_ref, id_ref, b_ref, w_ref, out_ref):
    od = od_ref[0] + od_ref[1]
    idg = id_ref[0] + id_ref[1]
    nsrc = jnp.where(od > 0, lax.rsqrt(jnp.maximum(od, 1.0)), 0.0)
    ndst = jnp.where(idg > 0, lax.rsqrt(jnp.maximum(idg, 1.0)), 0.0)
    agg = jnp.concatenate([agg_ref[0], agg_ref[1]], axis=1)
    h = jnp.maximum(agg * ndst[:, None] + b_ref[...], 0.0)
    y = jnp.dot(h * nsrc[:, None], w_ref[...],
                preferred_element_type=jnp.float32)
    out_ref[0] = y[:, :FH]
    out_ref[1] = y[:, FH:]


def _mm2(agg, odp, idp, b1p, w2p):
    return pl.pallas_call(
        _mm2_body,
        grid=(NBLK,),
        in_specs=[
            pl.BlockSpec((NC, RB, FH), lambda i: (0, i, 0)),
            pl.BlockSpec((NC, RB), lambda i: (0, i)),
            pl.BlockSpec((NC, RB), lambda i: (0, i)),
            pl.BlockSpec((1, F), lambda i: (0, 0)),
            pl.BlockSpec((F, F), lambda i: (0, 0)),
        ],
        out_specs=pl.BlockSpec((NC, RB, FH), lambda i: (0, i, 0)),
        out_shape=jax.ShapeDtypeStruct((NC, NPAD, FH), jnp.float32),
    )(agg, odp, idp, b1p, w2p)


def _fin_body(agg_ref, id_ref, b_ref, wfc_ref, out_ref):
    i = pl.program_id(0)
    idg = id_ref[0] + id_ref[1]
    ndst = jnp.where(idg > 0, lax.rsqrt(jnp.maximum(idg, 1.0)), 0.0)
    agg = jnp.concatenate([agg_ref[0], agg_ref[1]], axis=1)
    h = jnp.maximum(agg * ndst[:, None] + b_ref[...], 0.0)
    rows = i * RB + lax.broadcasted_iota(jnp.int32, (RB, 1), 0)
    h = jnp.where(rows < N, h, 0.0)
    part = jnp.sum(h * wfc_ref[...], dtype=jnp.float32) / N

    @pl.when(i == 0)
    def _():
        out_ref[0, 0] = 0.0

    out_ref[0, 0] += part


def _fin(agg, idp, b2p, wfcp):
    return pl.pallas_call(
        _fin_body,
        grid=(NBLK,),
        in_specs=[
            pl.BlockSpec((NC, RB, FH), lambda i: (0, i, 0)),
            pl.BlockSpec((NC, RB), lambda i: (0, i)),
            pl.BlockSpec((1, F), lambda i: (0, 0)),
            pl.BlockSpec((1, F), lambda i: (0, 0)),
        ],
        out_specs=pl.BlockSpec((1, 1), lambda i: (0, 0),
                               memory_space=pltpu.SMEM),
        out_shape=jax.ShapeDtypeStruct((1, 1), jnp.float32),
    )(agg, idp, b2p, wfcp)


def kernel(x, edge_index, W1, b1, W2, b2, W_fc, b_fc):
    src = edge_index[0].astype(jnp.int32)
    dst = edge_index[1].astype(jnp.int32)
    npad_e = EPAD - E
    padr = (jnp.arange(npad_e, dtype=jnp.int32) % (NPAD - N)) + N
    srcb = jnp.concatenate([src, padr]).reshape(NS, EC, CHUNK)
    dstb = jnp.concatenate([dst, padr]).reshape(NS, EC, CHUNK)
    srcg = jnp.stack([srcb, srcb + NPAD])

    xp = jnp.pad(x, ((0, NPAD - N), (0, 0)))
    w1p = jnp.pad(W1, ((0, 0), (0, F - H)))
    w2p = jnp.pad(W2, ((0, F - H), (0, F - H)))
    b1p = jnp.pad(b1, (0, F - H)).reshape(1, F)
    b2p = jnp.pad(b2, (0, F - H)).reshape(1, F)
    wfcp = jnp.pad(W_fc, ((0, 0), (0, F - H)))

    ones_h = jnp.ones((CHUNK,), jnp.float32)
    zeros_1d = jnp.zeros((RPT,), jnp.float32)
    zeros_2d = jnp.zeros((RPT, FH), jnp.float32)

    deg_call, seg_call = _sc_calls()
    deg = deg_call(srcg, dstb, ones_h, zeros_1d)
    odp = deg[:, 0, :]
    idp = deg[:, 1, :]

    t1 = _mm1(xp, odp, w1p).reshape(NC * NPAD, FH)
    agg1 = seg_call(srcg, dstb, t1, zeros_2d)
    t2 = _mm2(agg1, odp, idp, b1p, w2p).reshape(NC * NPAD, FH)
    agg2 = seg_call(srcg, dstb, t2, zeros_2d)
    out = _fin(agg2, idp, b2p, wfcp)
    return out.reshape(1) + b_fc


# speedup vs baseline: 8.2285x; 1.2866x over previous
import functools

import jax
import jax.numpy as jnp
from jax import lax
from jax.experimental import pallas as pl
from jax.experimental.pallas import tpu as pltpu
from jax.experimental.pallas import tpu_sc as plsc

N, E, DIN, H = 10000, 320000, 128, 246

NC, NS, LANES = 2, 16, 16
NPAD = 10240
F = 256
FH = F // NC
CHUNK = 128
EC = 160
EPAD = NS * EC * CHUNK
ECH = EC // NC
BCH = 16
RPT = NPAD // NS
RB = 640
NBLK = NPAD // RB


def _deg_body(srcg, dstg, ones_h, zeros_h, out, sidx, didx, ones_v,
              od_sh, id_sh):
    c = lax.axis_index("c")
    s = lax.axis_index("s")
    pltpu.sync_copy(zeros_h, od_sh.at[pl.ds(s * RPT, RPT)])
    pltpu.sync_copy(zeros_h, id_sh.at[pl.ds(s * RPT, RPT)])
    pltpu.sync_copy(ones_h, ones_v)
    pltpu.sync_copy(srcg.at[0, s, pl.ds(c * ECH, ECH)], sidx)
    pltpu.sync_copy(dstg.at[s, pl.ds(c * ECH, ECH)], didx)
    plsc.subcore_barrier()

    def body(j, carry):
        pltpu.sync_copy(ones_v, od_sh.at[sidx.at[j]], add=True)
        pltpu.sync_copy(ones_v, id_sh.at[didx.at[j]], add=True)
        return carry

    lax.fori_loop(0, ECH, body, 0)
    plsc.subcore_barrier()
    pltpu.sync_copy(od_sh.at[pl.ds(s * RPT, RPT)],
                    out.at[c, 0, pl.ds(s * RPT, RPT)])
    pltpu.sync_copy(id_sh.at[pl.ds(s * RPT, RPT)],
                    out.at[c, 1, pl.ds(s * RPT, RPT)])


@functools.lru_cache(maxsize=None)
def _sc_calls():
    mesh = plsc.VectorSubcoreMesh(core_axis_name="c", subcore_axis_name="s",
                                  num_cores=NC, num_subcores=NS)
    deg_call = pl.kernel(
        _deg_body,
        out_type=jax.ShapeDtypeStruct((NC, 2, NPAD), jnp.float32),
        mesh=mesh,
        scratch_types=[
            pltpu.VMEM((ECH, CHUNK), jnp.int32),
            pltpu.VMEM((ECH, CHUNK), jnp.int32),
            pltpu.VMEM((CHUNK,), jnp.float32),
            pltpu.VMEM_SHARED((NPAD,), jnp.float32),
            pltpu.VMEM_SHARED((NPAD,), jnp.float32),
        ],
    )
    seg_call = pl.kernel(
        _seg_body,
        out_type=jax.ShapeDtypeStruct((NC, NPAD, FH), jnp.float32),
        mesh=mesh,
        scratch_types=[
            pltpu.VMEM((BCH, CHUNK), jnp.int32),
            pltpu.VMEM((BCH, CHUNK), jnp.int32),
            pltpu.VMEM((CHUNK, FH), jnp.float32),
            pltpu.VMEM((CHUNK, FH), jnp.float32),
            pltpu.VMEM_SHARED((NPAD, FH), jnp.float32),
            pltpu.SemaphoreType.DMA,
            pltpu.SemaphoreType.DMA,
        ],
    )
    return deg_call, seg_call


def _seg_body(srcg, dstg, table, zeros_h, out, sidx, didx, rows0, rows1,
              acc_sh, gsem0, gsem1):
    c = lax.axis_index("c")
    s = lax.axis_index("s")
    pltpu.sync_copy(zeros_h, acc_sh.at[pl.ds(s * RPT, RPT)])
    plsc.subcore_barrier()

    def blk(b, carry):
        pltpu.sync_copy(srcg.at[c, s, pl.ds(b * BCH, BCH)], sidx)
        pltpu.sync_copy(dstg.at[s, pl.ds(b * BCH, BCH)], didx)
        pltpu.async_copy(table.at[sidx.at[0]], rows0, gsem0)

        def pair(j2, carry2):
            a = 2 * j2
            pltpu.make_async_copy(table.at[sidx.at[a]], rows0, gsem0).wait()
            pltpu.async_copy(table.at[sidx.at[a + 1]], rows1, gsem1)
            pltpu.sync_copy(rows0, acc_sh.at[didx.at[a]], add=True)
            pltpu.make_async_copy(table.at[sidx.at[a]], rows1, gsem1).wait()

            @pl.when(j2 < BCH // 2 - 1)
            def _():
                pltpu.async_copy(table.at[sidx.at[a + 2]], rows0, gsem0)

            pltpu.sync_copy(rows1, acc_sh.at[didx.at[a + 1]], add=True)
            return carry2

        lax.fori_loop(0, BCH // 2, pair, 0)
        return carry

    lax.fori_loop(0, EC // BCH, blk, 0)
    plsc.subcore_barrier()
    pltpu.sync_copy(acc_sh.at[pl.ds(s * RPT, RPT)],
                    out.at[c, pl.ds(s * RPT, RPT)])


def _mm1_body(x_ref, od_ref, w_ref, out_ref):
    od = od_ref[0] + od_ref[1]
    nsrc = jnp.where(od > 0, lax.rsqrt(jnp.maximum(od, 1.0)), 0.0)
    y = jnp.dot(x_ref[...] * nsrc[:, None], w_ref[...],
                preferred_element_type=jnp.float32)
    out_ref[0] = y[:, :FH]
    out_ref[1] = y[:, FH:]


def _mm1(xp, odp, w1p):
    return pl.pallas_call(
        _mm1_body,
        grid=(NBLK,),
        in_specs=[
            pl.BlockSpec((RB, DIN), lambda i: (i, 0)),
            pl.BlockSpec((NC, RB), lambda i: (0, i)),
            pl.BlockSpec((DIN, F), lambda i: (0, 0)),
        ],
        out_specs=pl.BlockSpec((NC, RB, FH), lambda i: (0, i, 0)),
        out_shape=jax.ShapeDtypeStruct((NC, NPAD, FH), jnp.float32),
    )(xp, odp, w1p)


def _mm2_body(agg_ref, od_ref, id_ref, b_ref, w_ref, out_ref):
    od = od_ref[0] + od_ref[1]
    idg = id_ref[0] + id_ref[1]
    nsrc = jnp.where(od > 0, lax.rsqrt(jnp.maximum(od, 1.0)), 0.0)
    ndst = jnp.where(idg > 0, lax.rsqrt(jnp.maximum(idg, 1.0)), 0.0)
    agg = jnp.concatenate([agg_ref[0], agg_ref[1]], axis=1)
    h = jnp.maximum(agg * ndst[:, None] + b_ref[...], 0.0)
    y = jnp.dot(h * nsrc[:, None], w_ref[...],
                preferred_element_type=jnp.float32)
    out_ref[0] = y[:, :FH]
    out_ref[1] = y[:, FH:]


def _mm2(agg, odp, idp, b1p, w2p):
    return pl.pallas_call(
        _mm2_body,
        grid=(NBLK,),
        in_specs=[
            pl.BlockSpec((NC, RB, FH), lambda i: (0, i, 0)),
            pl.BlockSpec((NC, RB), lambda i: (0, i)),
            pl.BlockSpec((NC, RB), lambda i: (0, i)),
            pl.BlockSpec((1, F), lambda i: (0, 0)),
            pl.BlockSpec((F, F), lambda i: (0, 0)),
        ],
        out_specs=pl.BlockSpec((NC, RB, FH), lambda i: (0, i, 0)),
        out_shape=jax.ShapeDtypeStruct((NC, NPAD, FH), jnp.float32),
    )(agg, odp, idp, b1p, w2p)


def _fin_body(agg_ref, id_ref, b_ref, wfc_ref, out_ref):
    i = pl.program_id(0)
    idg = id_ref[0] + id_ref[1]
    ndst = jnp.where(idg > 0, lax.rsqrt(jnp.maximum(idg, 1.0)), 0.0)
    agg = jnp.concatenate([agg_ref[0], agg_ref[1]], axis=1)
    h = jnp.maximum(agg * ndst[:, None] + b_ref[...], 0.0)
    rows = i * RB + lax.broadcasted_iota(jnp.int32, (RB, 1), 0)
    h = jnp.where(rows < N, h, 0.0)
    part = jnp.sum(h * wfc_ref[...], dtype=jnp.float32) / N

    @pl.when(i == 0)
    def _():
        out_ref[0, 0] = 0.0

    out_ref[0, 0] += part


def _fin(agg, idp, b2p, wfcp):
    return pl.pallas_call(
        _fin_body,
        grid=(NBLK,),
        in_specs=[
            pl.BlockSpec((NC, RB, FH), lambda i: (0, i, 0)),
            pl.BlockSpec((NC, RB), lambda i: (0, i)),
            pl.BlockSpec((1, F), lambda i: (0, 0)),
            pl.BlockSpec((1, F), lambda i: (0, 0)),
        ],
        out_specs=pl.BlockSpec((1, 1), lambda i: (0, 0),
                               memory_space=pltpu.SMEM),
        out_shape=jax.ShapeDtypeStruct((1, 1), jnp.float32),
    )(agg, idp, b2p, wfcp)


def kernel(x, edge_index, W1, b1, W2, b2, W_fc, b_fc):
    src = edge_index[0].astype(jnp.int32)
    dst = edge_index[1].astype(jnp.int32)
    npad_e = EPAD - E
    padr = (jnp.arange(npad_e, dtype=jnp.int32) % (NPAD - N)) + N
    srcb = jnp.concatenate([src, padr]).reshape(NS, EC, CHUNK)
    dstb = jnp.concatenate([dst, padr]).reshape(NS, EC, CHUNK)
    srcg = jnp.stack([srcb, srcb + NPAD])

    xp = jnp.pad(x, ((0, NPAD - N), (0, 0)))
    w1p = jnp.pad(W1, ((0, 0), (0, F - H)))
    w2p = jnp.pad(W2, ((0, F - H), (0, F - H)))
    b1p = jnp.pad(b1, (0, F - H)).reshape(1, F)
    b2p = jnp.pad(b2, (0, F - H)).reshape(1, F)
    wfcp = jnp.pad(W_fc, ((0, 0), (0, F - H)))

    ones_h = jnp.ones((CHUNK,), jnp.float32)
    zeros_1d = jnp.zeros((RPT,), jnp.float32)
    zeros_2d = jnp.zeros((RPT, FH), jnp.float32)

    deg_call, seg_call = _sc_calls()
    deg = deg_call(srcg, dstb, ones_h, zeros_1d)
    odp = deg[:, 0, :]
    idp = deg[:, 1, :]

    t1 = _mm1(xp, odp, w1p).reshape(NC * NPAD, FH)
    agg1 = seg_call(srcg, dstb, t1, zeros_2d)
    t2 = _mm2(agg1, odp, idp, b1p, w2p).reshape(NC * NPAD, FH)
    agg2 = seg_call(srcg, dstb, t2, zeros_2d)
    out = _fin(agg2, idp, b2p, wfcp)
    return out.reshape(1) + b_fc


# speedup vs baseline: 8.8205x; 1.0719x over previous
import functools

import jax
import jax.numpy as jnp
from jax import lax
from jax.experimental import pallas as pl
from jax.experimental.pallas import tpu as pltpu
from jax.experimental.pallas import tpu_sc as plsc

N, E, DIN, H = 10000, 320000, 128, 246

NC, NS, LANES = 2, 16, 16
NPAD = 10240
F = 256
FH = F // NC
CHUNK = 80
EC = 256
EPAD = NS * EC * CHUNK
ECH = EC // NC
BCH = 32
RPT = NPAD // NS
RB = 640
NBLK = NPAD // RB


def _deg_body(srcg, dstg, ones_h, zeros_h, out, sidx, didx, ones_v,
              od_sh, id_sh):
    c = lax.axis_index("c")
    s = lax.axis_index("s")
    pltpu.sync_copy(zeros_h, od_sh.at[pl.ds(s * RPT, RPT)])
    pltpu.sync_copy(zeros_h, id_sh.at[pl.ds(s * RPT, RPT)])
    pltpu.sync_copy(ones_h, ones_v)
    pltpu.sync_copy(srcg.at[0, s, pl.ds(c * ECH, ECH)], sidx)
    pltpu.sync_copy(dstg.at[s, pl.ds(c * ECH, ECH)], didx)
    plsc.subcore_barrier()

    def body(j, carry):
        pltpu.sync_copy(ones_v, od_sh.at[sidx.at[j]], add=True)
        pltpu.sync_copy(ones_v, id_sh.at[didx.at[j]], add=True)
        return carry

    lax.fori_loop(0, ECH, body, 0)
    plsc.subcore_barrier()
    pltpu.sync_copy(od_sh.at[pl.ds(s * RPT, RPT)],
                    out.at[c, 0, pl.ds(s * RPT, RPT)])
    pltpu.sync_copy(id_sh.at[pl.ds(s * RPT, RPT)],
                    out.at[c, 1, pl.ds(s * RPT, RPT)])


@functools.lru_cache(maxsize=None)
def _sc_calls():
    mesh = plsc.VectorSubcoreMesh(core_axis_name="c", subcore_axis_name="s",
                                  num_cores=NC, num_subcores=NS)
    deg_call = pl.kernel(
        _deg_body,
        out_type=jax.ShapeDtypeStruct((NC, 2, NPAD), jnp.float32),
        mesh=mesh,
        scratch_types=[
            pltpu.VMEM((ECH, CHUNK), jnp.int32),
            pltpu.VMEM((ECH, CHUNK), jnp.int32),
            pltpu.VMEM((CHUNK,), jnp.float32),
            pltpu.VMEM_SHARED((NPAD,), jnp.float32),
            pltpu.VMEM_SHARED((NPAD,), jnp.float32),
        ],
    )
    seg_call = pl.kernel(
        _seg_body,
        out_type=jax.ShapeDtypeStruct((NC, NPAD, FH), jnp.float32),
        mesh=mesh,
        scratch_types=[
            pltpu.VMEM((BCH, CHUNK), jnp.int32),
            pltpu.VMEM((BCH, CHUNK), jnp.int32),
            pltpu.VMEM((CHUNK, FH), jnp.float32),
            pltpu.VMEM((CHUNK, FH), jnp.float32),
            pltpu.VMEM((CHUNK, FH), jnp.float32),
            pltpu.VMEM((CHUNK, FH), jnp.float32),
            pltpu.VMEM_SHARED((NPAD, FH), jnp.float32),
            pltpu.SemaphoreType.DMA,
            pltpu.SemaphoreType.DMA,
            pltpu.SemaphoreType.DMA,
            pltpu.SemaphoreType.DMA,
            pltpu.SemaphoreType.DMA,
            pltpu.SemaphoreType.DMA,
            pltpu.SemaphoreType.DMA,
            pltpu.SemaphoreType.DMA,
        ],
    )
    return deg_call, seg_call


def _seg_body(srcg, dstg, table, zeros_h, out, sidx, didx,
              rows0, rows1, rows2, rows3, acc_sh,
              g0, g1, g2, g3, s0, s1, s2, s3):
    c = lax.axis_index("c")
    s = lax.axis_index("s")
    rows = (rows0, rows1, rows2, rows3)
    gsem = (g0, g1, g2, g3)
    ssem = (s0, s1, s2, s3)
    pltpu.sync_copy(zeros_h, acc_sh.at[pl.ds(s * RPT, RPT)])
    plsc.subcore_barrier()

    def blk(b, carry):
        pltpu.sync_copy(srcg.at[c, s, pl.ds(b * BCH, BCH)], sidx)
        pltpu.sync_copy(dstg.at[s, pl.ds(b * BCH, BCH)], didx)
        pltpu.async_copy(table.at[sidx.at[0]], rows0, gsem[0])
        pltpu.async_copy(table.at[sidx.at[1]], rows1, gsem[1])

        def quad(q, carry2):
            for bb in range(4):
                ch = 4 * q + bb
                nb = (bb + 2) % 4
                pltpu.make_async_copy(table.at[sidx.at[ch]], rows[bb],
                                      gsem[bb]).wait()
                pltpu.async_copy(rows[bb], acc_sh.at[didx.at[ch]], ssem[bb],
                                 add=True)

                @pl.when(ch >= 2)
                def _():
                    pltpu.make_async_copy(rows[nb], acc_sh.at[didx.at[ch]],
                                          ssem[nb]).wait()

                @pl.when(ch + 2 < BCH)
                def _():
                    pltpu.async_copy(table.at[sidx.at[ch + 2]], rows[nb],
                                     gsem[nb])

            return carry2

        lax.fori_loop(0, BCH // 4, quad, 0)
        pltpu.make_async_copy(rows2, acc_sh.at[didx.at[0]], ssem[2]).wait()
        pltpu.make_async_copy(rows3, acc_sh.at[didx.at[0]], ssem[3]).wait()
        return carry

    lax.fori_loop(0, EC // BCH, blk, 0)
    plsc.subcore_barrier()
    pltpu.sync_copy(acc_sh.at[pl.ds(s * RPT, RPT)],
                    out.at[c, pl.ds(s * RPT, RPT)])


def _mm1_body(x_ref, od_ref, w_ref, out_ref):
    od = od_ref[0] + od_ref[1]
    nsrc = jnp.where(od > 0, lax.rsqrt(jnp.maximum(od, 1.0)), 0.0)
    y = jnp.dot(x_ref[...] * nsrc[:, None], w_ref[...],
                preferred_element_type=jnp.float32)
    out_ref[0] = y[:, :FH]
    out_ref[1] = y[:, FH:]


def _mm1(xp, odp, w1p):
    return pl.pallas_call(
        _mm1_body,
        grid=(NBLK,),
        in_specs=[
            pl.BlockSpec((RB, DIN), lambda i: (i, 0)),
            pl.BlockSpec((NC, RB), lambda i: (0, i)),
            pl.BlockSpec((DIN, F), lambda i: (0, 0)),
        ],
        out_specs=pl.BlockSpec((NC, RB, FH), lambda i: (0, i, 0)),
        out_shape=jax.ShapeDtypeStruct((NC, NPAD, FH), jnp.float32),
    )(xp, odp, w1p)


def _mm2_body(agg_ref, od_ref, id_ref, b_ref, w_ref, out_ref):
    od = od_ref[0] + od_ref[1]
    idg = id_ref[0] + id_ref[1]
    nsrc = jnp.where(od > 0, lax.rsqrt(jnp.maximum(od, 1.0)), 0.0)
    ndst = jnp.where(idg > 0, lax.rsqrt(jnp.maximum(idg, 1.0)), 0.0)
    agg = jnp.concatenate([agg_ref[0], agg_ref[1]], axis=1)
    h = jnp.maximum(agg * ndst[:, None] + b_ref[...], 0.0)
    y = jnp.dot(h * nsrc[:, None], w_ref[...],
                preferred_element_type=jnp.float32)
    out_ref[0] = y[:, :FH]
    out_ref[1] = y[:, FH:]


def _mm2(agg, odp, idp, b1p, w2p):
    return pl.pallas_call(
        _mm2_body,
        grid=(NBLK,),
        in_specs=[
            pl.BlockSpec((NC, RB, FH), lambda i: (0, i, 0)),
            pl.BlockSpec((NC, RB), lambda i: (0, i)),
            pl.BlockSpec((NC, RB), lambda i: (0, i)),
            pl.BlockSpec((1, F), lambda i: (0, 0)),
            pl.BlockSpec((F, F), lambda i: (0, 0)),
        ],
        out_specs=pl.BlockSpec((NC, RB, FH), lambda i: (0, i, 0)),
        out_shape=jax.ShapeDtypeStruct((NC, NPAD, FH), jnp.float32),
    )(agg, odp, idp, b1p, w2p)


def _fin_body(agg_ref, id_ref, b_ref, wfc_ref, out_ref):
    i = pl.program_id(0)
    idg = id_ref[0] + id_ref[1]
    ndst = jnp.where(idg > 0, lax.rsqrt(jnp.maximum(idg, 1.0)), 0.0)
    agg = jnp.concatenate([agg_ref[0], agg_ref[1]], axis=1)
    h = jnp.maximum(agg * ndst[:, None] + b_ref[...], 0.0)
    rows = i * RB + lax.broadcasted_iota(jnp.int32, (RB, 1), 0)
    h = jnp.where(rows < N, h, 0.0)
    part = jnp.sum(h * wfc_ref[...], dtype=jnp.float32) / N

    @pl.when(i == 0)
    def _():
        out_ref[0, 0] = 0.0

    out_ref[0, 0] += part


def _fin(agg, idp, b2p, wfcp):
    return pl.pallas_call(
        _fin_body,
        grid=(NBLK,),
        in_specs=[
            pl.BlockSpec((NC, RB, FH), lambda i: (0, i, 0)),
            pl.BlockSpec((NC, RB), lambda i: (0, i)),
            pl.BlockSpec((1, F), lambda i: (0, 0)),
            pl.BlockSpec((1, F), lambda i: (0, 0)),
        ],
        out_specs=pl.BlockSpec((1, 1), lambda i: (0, 0),
                               memory_space=pltpu.SMEM),
        out_shape=jax.ShapeDtypeStruct((1, 1), jnp.float32),
    )(agg, idp, b2p, wfcp)


def kernel(x, edge_index, W1, b1, W2, b2, W_fc, b_fc):
    src = edge_index[0].astype(jnp.int32)
    dst = edge_index[1].astype(jnp.int32)
    npad_e = EPAD - E
    padr = (jnp.arange(npad_e, dtype=jnp.int32) % (NPAD - N)) + N
    srcb = jnp.concatenate([src, padr]).reshape(NS, EC, CHUNK)
    dstb = jnp.concatenate([dst, padr]).reshape(NS, EC, CHUNK)
    srcg = jnp.stack([srcb, srcb + NPAD])

    xp = jnp.pad(x, ((0, NPAD - N), (0, 0)))
    w1p = jnp.pad(W1, ((0, 0), (0, F - H)))
    w2p = jnp.pad(W2, ((0, F - H), (0, F - H)))
    b1p = jnp.pad(b1, (0, F - H)).reshape(1, F)
    b2p = jnp.pad(b2, (0, F - H)).reshape(1, F)
    wfcp = jnp.pad(W_fc, ((0, 0), (0, F - H)))

    ones_h = jnp.ones((CHUNK,), jnp.float32)
    zeros_1d = jnp.zeros((RPT,), jnp.float32)
    zeros_2d = jnp.zeros((RPT, FH), jnp.float32)

    deg_call, seg_call = _sc_calls()
    deg = deg_call(srcg, dstb, ones_h, zeros_1d)
    odp = deg[:, 0, :]
    idp = deg[:, 1, :]

    t1 = _mm1(xp, odp, w1p).reshape(NC * NPAD, FH)
    agg1 = seg_call(srcg, dstb, t1, zeros_2d)
    t2 = _mm2(agg1, odp, idp, b1p, w2p).reshape(NC * NPAD, FH)
    agg2 = seg_call(srcg, dstb, t2, zeros_2d)
    out = _fin(agg2, idp, b2p, wfcp)
    return out.reshape(1) + b_fc


# speedup vs baseline: 9.2755x; 1.0516x over previous
import functools

import jax
import jax.numpy as jnp
from jax import lax
from jax.experimental import pallas as pl
from jax.experimental.pallas import tpu as pltpu
from jax.experimental.pallas import tpu_sc as plsc

N, E, DIN, H = 10000, 320000, 128, 246

NC, NS, LANES = 2, 16, 16
NPAD = 10240
F = 256
FH = F // NC
CHUNK = 80
EC = 256
EPAD = NS * EC * CHUNK
ECH = EC // NC
BCH = 16
RPT = NPAD // NS
RB = 640
NBLK = NPAD // RB


def _deg_body(srcg, dstg, ones_h, zeros_h, out, sidx, didx, ones_v,
              od_sh, id_sh):
    c = lax.axis_index("c")
    s = lax.axis_index("s")
    pltpu.sync_copy(zeros_h, od_sh.at[pl.ds(s * RPT, RPT)])
    pltpu.sync_copy(zeros_h, id_sh.at[pl.ds(s * RPT, RPT)])
    pltpu.sync_copy(ones_h, ones_v)
    pltpu.sync_copy(srcg.at[0, s, pl.ds(c * ECH, ECH)], sidx)
    pltpu.sync_copy(dstg.at[s, pl.ds(c * ECH, ECH)], didx)
    plsc.subcore_barrier()

    def body(j, carry):
        pltpu.sync_copy(ones_v, od_sh.at[sidx.at[j]], add=True)
        pltpu.sync_copy(ones_v, id_sh.at[didx.at[j]], add=True)
        return carry

    lax.fori_loop(0, ECH, body, 0)
    plsc.subcore_barrier()
    pltpu.sync_copy(od_sh.at[pl.ds(s * RPT, RPT)],
                    out.at[c, 0, pl.ds(s * RPT, RPT)])
    pltpu.sync_copy(id_sh.at[pl.ds(s * RPT, RPT)],
                    out.at[c, 1, pl.ds(s * RPT, RPT)])


@functools.lru_cache(maxsize=None)
def _sc_calls():
    mesh = plsc.VectorSubcoreMesh(core_axis_name="c", subcore_axis_name="s",
                                  num_cores=NC, num_subcores=NS)
    deg_call = pl.kernel(
        _deg_body,
        out_type=jax.ShapeDtypeStruct((NC, 2, NPAD), jnp.float32),
        mesh=mesh,
        scratch_types=[
            pltpu.VMEM((ECH, CHUNK), jnp.int32),
            pltpu.VMEM((ECH, CHUNK), jnp.int32),
            pltpu.VMEM((CHUNK,), jnp.float32),
            pltpu.VMEM_SHARED((NPAD,), jnp.float32),
            pltpu.VMEM_SHARED((NPAD,), jnp.float32),
        ],
    )
    seg_call = pl.kernel(
        _seg_body,
        out_type=jax.ShapeDtypeStruct((NC, NPAD, FH), jnp.float32),
        mesh=mesh,
        scratch_types=[
            pltpu.VMEM((BCH, CHUNK), jnp.int32),
            pltpu.VMEM((BCH, CHUNK), jnp.int32),
            pltpu.VMEM((BCH, CHUNK), jnp.int32),
            pltpu.VMEM((BCH, CHUNK), jnp.int32),
            pltpu.VMEM((CHUNK, FH), jnp.float32),
            pltpu.VMEM((CHUNK, FH), jnp.float32),
            pltpu.VMEM((CHUNK, FH), jnp.float32),
            pltpu.VMEM((CHUNK, FH), jnp.float32),
            pltpu.VMEM_SHARED((NPAD, FH), jnp.float32),
            pltpu.SemaphoreType.DMA,
            pltpu.SemaphoreType.DMA,
            pltpu.SemaphoreType.DMA,
            pltpu.SemaphoreType.DMA,
            pltpu.SemaphoreType.DMA,
            pltpu.SemaphoreType.DMA,
            pltpu.SemaphoreType.DMA,
            pltpu.SemaphoreType.DMA,
            pltpu.SemaphoreType.DMA,
        ],
    )
    return deg_call, seg_call


def _seg_body(srcg, dstg, table, zeros_h, out, sidxA, didxA, sidxB, didxB,
              rows0, rows1, rows2, rows3, acc_sh,
              g0, g1, g2, g3, s0, s1, s2, s3, isem):
    c = lax.axis_index("c")
    s = lax.axis_index("s")
    rows = (rows0, rows1, rows2, rows3)
    gsem = (g0, g1, g2, g3)
    ssem = (s0, s1, s2, s3)
    nblk = EC // BCH
    pltpu.sync_copy(zeros_h, acc_sh.at[pl.ds(s * RPT, RPT)])
    plsc.subcore_barrier()

    pltpu.sync_copy(srcg.at[c, s, pl.ds(0, BCH)], sidxA)
    pltpu.sync_copy(dstg.at[s, pl.ds(0, BCH)], didxA)
    pltpu.async_copy(table.at[sidxA.at[0]], rows0, gsem[0])
    pltpu.async_copy(table.at[sidxA.at[1]], rows1, gsem[1])

    def process(b, cur_s, cur_d, nxt_s, nxt_d):
        def quad(q, carry):
            for bb in range(4):
                ch = 4 * q + bb
                g = b * BCH + ch
                nb = (bb + 2) % 4
                pltpu.make_async_copy(table.at[cur_s.at[ch]], rows[bb],
                                      gsem[bb]).wait()
                pltpu.async_copy(rows[bb], acc_sh.at[cur_d.at[ch]], ssem[bb],
                                 add=True)
                if bb == 2:
                    @pl.when((ch == 2) & (b + 1 < nblk))
                    def _():
                        pltpu.async_copy(srcg.at[c, s,
                                                 pl.ds((b + 1) * BCH, BCH)],
                                         nxt_s, isem)
                        pltpu.async_copy(dstg.at[s, pl.ds((b + 1) * BCH, BCH)],
                                         nxt_d, isem)

                @pl.when(g >= 2)
                def _():
                    pltpu.make_async_copy(rows[nb], acc_sh.at[cur_d.at[ch]],
                                          ssem[nb]).wait()

                @pl.when(ch + 2 < BCH)
                def _():
                    pltpu.async_copy(table.at[cur_s.at[ch + 2]], rows[nb],
                                     gsem[nb])

                @pl.when((ch + 2 >= BCH) & (b + 1 < nblk))
                def _():
                    @pl.when(ch == BCH - 2)
                    def _():
                        pltpu.make_async_copy(
                            srcg.at[c, s, pl.ds(0, BCH)], nxt_s, isem).wait()
                        pltpu.make_async_copy(
                            dstg.at[s, pl.ds(0, BCH)], nxt_d, isem).wait()

                    pltpu.async_copy(table.at[nxt_s.at[ch + 2 - BCH]],
                                     rows[nb], gsem[nb])

            return carry

        lax.fori_loop(0, BCH // 4, quad, 0)

    def pairs(pp, carry):
        process(2 * pp, sidxA, didxA, sidxB, didxB)
        process(2 * pp + 1, sidxB, didxB, sidxA, didxA)
        return carry

    lax.fori_loop(0, nblk // 2, pairs, 0)
    pltpu.make_async_copy(rows2, acc_sh.at[didxA.at[0]], ssem[2]).wait()
    pltpu.make_async_copy(rows3, acc_sh.at[didxA.at[0]], ssem[3]).wait()

    plsc.subcore_barrier()
    pltpu.sync_copy(acc_sh.at[pl.ds(s * RPT, RPT)],
                    out.at[c, pl.ds(s * RPT, RPT)])


def _mm1_body(x_ref, od_ref, w_ref, out_ref):
    od = od_ref[0] + od_ref[1]
    nsrc = jnp.where(od > 0, lax.rsqrt(jnp.maximum(od, 1.0)), 0.0)
    y = jnp.dot(x_ref[...] * nsrc[:, None], w_ref[...],
                preferred_element_type=jnp.float32)
    out_ref[0] = y[:, :FH]
    out_ref[1] = y[:, FH:]


def _mm1(xp, odp, w1p):
    return pl.pallas_call(
        _mm1_body,
        grid=(NBLK,),
        in_specs=[
            pl.BlockSpec((RB, DIN), lambda i: (i, 0)),
            pl.BlockSpec((NC, RB), lambda i: (0, i)),
            pl.BlockSpec((DIN, F), lambda i: (0, 0)),
        ],
        out_specs=pl.BlockSpec((NC, RB, FH), lambda i: (0, i, 0)),
        out_shape=jax.ShapeDtypeStruct((NC, NPAD, FH), jnp.float32),
    )(xp, odp, w1p)


def _mm2_body(agg_ref, od_ref, id_ref, b_ref, w_ref, out_ref):
    od = od_ref[0] + od_ref[1]
    idg = id_ref[0] + id_ref[1]
    nsrc = jnp.where(od > 0, lax.rsqrt(jnp.maximum(od, 1.0)), 0.0)
    ndst = jnp.where(idg > 0, lax.rsqrt(jnp.maximum(idg, 1.0)), 0.0)
    agg = jnp.concatenate([agg_ref[0], agg_ref[1]], axis=1)
    h = jnp.maximum(agg * ndst[:, None] + b_ref[...], 0.0)
    y = jnp.dot(h * nsrc[:, None], w_ref[...],
                preferred_element_type=jnp.float32)
    out_ref[0] = y[:, :FH]
    out_ref[1] = y[:, FH:]


def _mm2(agg, odp, idp, b1p, w2p):
    return pl.pallas_call(
        _mm2_body,
        grid=(NBLK,),
        in_specs=[
            pl.BlockSpec((NC, RB, FH), lambda i: (0, i, 0)),
            pl.BlockSpec((NC, RB), lambda i: (0, i)),
            pl.BlockSpec((NC, RB), lambda i: (0, i)),
            pl.BlockSpec((1, F), lambda i: (0, 0)),
            pl.BlockSpec((F, F), lambda i: (0, 0)),
        ],
        out_specs=pl.BlockSpec((NC, RB, FH), lambda i: (0, i, 0)),
        out_shape=jax.ShapeDtypeStruct((NC, NPAD, FH), jnp.float32),
    )(agg, odp, idp, b1p, w2p)


def _fin_body(agg_ref, id_ref, b_ref, wfc_ref, out_ref):
    i = pl.program_id(0)
    idg = id_ref[0] + id_ref[1]
    ndst = jnp.where(idg > 0, lax.rsqrt(jnp.maximum(idg, 1.0)), 0.0)
    agg = jnp.concatenate([agg_ref[0], agg_ref[1]], axis=1)
    h = jnp.maximum(agg * ndst[:, None] + b_ref[...], 0.0)
    rows = i * RB + lax.broadcasted_iota(jnp.int32, (RB, 1), 0)
    h = jnp.where(rows < N, h, 0.0)
    part = jnp.sum(h * wfc_ref[...], dtype=jnp.float32) / N

    @pl.when(i == 0)
    def _():
        out_ref[0, 0] = 0.0

    out_ref[0, 0] += part


def _fin(agg, idp, b2p, wfcp):
    return pl.pallas_call(
        _fin_body,
        grid=(NBLK,),
        in_specs=[
            pl.BlockSpec((NC, RB, FH), lambda i: (0, i, 0)),
            pl.BlockSpec((NC, RB), lambda i: (0, i)),
            pl.BlockSpec((1, F), lambda i: (0, 0)),
            pl.BlockSpec((1, F), lambda i: (0, 0)),
        ],
        out_specs=pl.BlockSpec((1, 1), lambda i: (0, 0),
                               memory_space=pltpu.SMEM),
        out_shape=jax.ShapeDtypeStruct((1, 1), jnp.float32),
    )(agg, idp, b2p, wfcp)


def kernel(x, edge_index, W1, b1, W2, b2, W_fc, b_fc):
    src = edge_index[0].astype(jnp.int32)
    dst = edge_index[1].astype(jnp.int32)
    npad_e = EPAD - E
    padr = (jnp.arange(npad_e, dtype=jnp.int32) % (NPAD - N)) + N
    srcb = jnp.concatenate([src, padr]).reshape(NS, EC, CHUNK)
    dstb = jnp.concatenate([dst, padr]).reshape(NS, EC, CHUNK)
    srcg = jnp.stack([srcb, srcb + NPAD])

    xp = jnp.pad(x, ((0, NPAD - N), (0, 0)))
    w1p = jnp.pad(W1, ((0, 0), (0, F - H)))
    w2p = jnp.pad(W2, ((0, F - H), (0, F - H)))
    b1p = jnp.pad(b1, (0, F - H)).reshape(1, F)
    b2p = jnp.pad(b2, (0, F - H)).reshape(1, F)
    wfcp = jnp.pad(W_fc, ((0, 0), (0, F - H)))

    ones_h = jnp.ones((CHUNK,), jnp.float32)
    zeros_1d = jnp.zeros((RPT,), jnp.float32)
    zeros_2d = jnp.zeros((RPT, FH), jnp.float32)

    deg_call, seg_call = _sc_calls()
    deg = deg_call(srcg, dstb, ones_h, zeros_1d)
    odp = deg[:, 0, :]
    idp = deg[:, 1, :]

    t1 = _mm1(xp, odp, w1p).reshape(NC * NPAD, FH)
    agg1 = seg_call(srcg, dstb, t1, zeros_2d)
    t2 = _mm2(agg1, odp, idp, b1p, w2p).reshape(NC * NPAD, FH)
    agg2 = seg_call(srcg, dstb, t2, zeros_2d)
    out = _fin(agg2, idp, b2p, wfcp)
    return out.reshape(1) + b_fc


# speedup vs baseline: 9.6209x; 1.0372x over previous
import functools

import jax
import jax.numpy as jnp
from jax import lax
from jax.experimental import pallas as pl
from jax.experimental.pallas import tpu as pltpu
from jax.experimental.pallas import tpu_sc as plsc

N, E, DIN, H = 10000, 320000, 128, 246

NC, NS, LANES = 2, 16, 16
NPAD = 10240
F = 256
FH = F // NC
CHUNK = 80
EC = 256
EPAD = NS * EC * CHUNK
ECH = EC // NC
BCH = 16
RPT = NPAD // NS
RB = 640
NBLK = NPAD // RB


def _deg_body(srcb, dstg, ones_h, zeros_h, out, sidx, didx, ones_v,
              od_sh, id_sh, o0, o1, o2, o3, i0, i1, i2, i3):
    c = lax.axis_index("c")
    s = lax.axis_index("s")
    osem = (o0, o1, o2, o3)
    isem = (i0, i1, i2, i3)
    pltpu.sync_copy(zeros_h, od_sh.at[pl.ds(s * RPT, RPT)])
    pltpu.sync_copy(zeros_h, id_sh.at[pl.ds(s * RPT, RPT)])
    pltpu.sync_copy(ones_h, ones_v)
    pltpu.sync_copy(srcb.at[s, pl.ds(c * ECH, ECH)], sidx)
    pltpu.sync_copy(dstg.at[s, pl.ds(c * ECH, ECH)], didx)
    plsc.subcore_barrier()

    def quad(q, carry):
        for bb in range(4):
            j = 4 * q + bb

            @pl.when(j >= 4)
            def _():
                pltpu.make_async_copy(ones_v, od_sh.at[sidx.at[j]],
                                      osem[bb]).wait()
                pltpu.make_async_copy(ones_v, id_sh.at[didx.at[j]],
                                      isem[bb]).wait()

            pltpu.async_copy(ones_v, od_sh.at[sidx.at[j]], osem[bb], add=True)
            pltpu.async_copy(ones_v, id_sh.at[didx.at[j]], isem[bb], add=True)
        return carry

    lax.fori_loop(0, ECH // 4, quad, 0)
    for bb in range(4):
        pltpu.make_async_copy(ones_v, od_sh.at[sidx.at[0]], osem[bb]).wait()
        pltpu.make_async_copy(ones_v, id_sh.at[didx.at[0]], isem[bb]).wait()
    plsc.subcore_barrier()
    pltpu.sync_copy(od_sh.at[pl.ds(s * RPT, RPT)],
                    out.at[c, 0, pl.ds(s * RPT, RPT)])
    pltpu.sync_copy(id_sh.at[pl.ds(s * RPT, RPT)],
                    out.at[c, 1, pl.ds(s * RPT, RPT)])


@functools.lru_cache(maxsize=None)
def _sc_calls():
    mesh = plsc.VectorSubcoreMesh(core_axis_name="c", subcore_axis_name="s",
                                  num_cores=NC, num_subcores=NS)
    deg_call = pl.kernel(
        _deg_body,
        out_type=jax.ShapeDtypeStruct((NC, 2, NPAD), jnp.float32),
        mesh=mesh,
        scratch_types=[
            pltpu.VMEM((ECH, CHUNK), jnp.int32),
            pltpu.VMEM((ECH, CHUNK), jnp.int32),
            pltpu.VMEM((CHUNK,), jnp.float32),
            pltpu.VMEM_SHARED((NPAD,), jnp.float32),
            pltpu.VMEM_SHARED((NPAD,), jnp.float32),
            pltpu.SemaphoreType.DMA,
            pltpu.SemaphoreType.DMA,
            pltpu.SemaphoreType.DMA,
            pltpu.SemaphoreType.DMA,
            pltpu.SemaphoreType.DMA,
            pltpu.SemaphoreType.DMA,
            pltpu.SemaphoreType.DMA,
            pltpu.SemaphoreType.DMA,
        ],
    )
    seg_call = pl.kernel(
        _seg_body,
        out_type=jax.ShapeDtypeStruct((NC, NPAD, FH), jnp.float32),
        mesh=mesh,
        scratch_types=[
            pltpu.VMEM((BCH, CHUNK), jnp.int32),
            pltpu.VMEM((BCH, CHUNK), jnp.int32),
            pltpu.VMEM((BCH, CHUNK), jnp.int32),
            pltpu.VMEM((BCH, CHUNK), jnp.int32),
            pltpu.VMEM((CHUNK, FH), jnp.float32),
            pltpu.VMEM((CHUNK, FH), jnp.float32),
            pltpu.VMEM((CHUNK, FH), jnp.float32),
            pltpu.VMEM((CHUNK, FH), jnp.float32),
            pltpu.VMEM_SHARED((NPAD, FH), jnp.float32),
            pltpu.SemaphoreType.DMA,
            pltpu.SemaphoreType.DMA,
            pltpu.SemaphoreType.DMA,
            pltpu.SemaphoreType.DMA,
            pltpu.SemaphoreType.DMA,
            pltpu.SemaphoreType.DMA,
            pltpu.SemaphoreType.DMA,
            pltpu.SemaphoreType.DMA,
            pltpu.SemaphoreType.DMA,
        ],
    )
    return deg_call, seg_call


def _seg_body(srcb, dstg, table, zeros_h, out, sidxA, didxA, sidxB, didxB,
              rows0, rows1, rows2, rows3, acc_sh,
              g0, g1, g2, g3, s0, s1, s2, s3, isem):
    c = lax.axis_index("c")
    s = lax.axis_index("s")
    rows = (rows0, rows1, rows2, rows3)
    gsem = (g0, g1, g2, g3)
    ssem = (s0, s1, s2, s3)
    nblk = EC // BCH
    pltpu.sync_copy(zeros_h, acc_sh.at[pl.ds(s * RPT, RPT)])
    plsc.subcore_barrier()

    pltpu.sync_copy(srcb.at[s, pl.ds(0, BCH)], sidxA)
    pltpu.sync_copy(dstg.at[s, pl.ds(0, BCH)], didxA)
    pltpu.async_copy(table.at[c].at[sidxA.at[0]], rows0, gsem[0])
    pltpu.async_copy(table.at[c].at[sidxA.at[1]], rows1, gsem[1])

    def process(b, cur_s, cur_d, nxt_s, nxt_d):
        def quad(q, carry):
            for bb in range(4):
                ch = 4 * q + bb
                g = b * BCH + ch
                nb = (bb + 2) % 4
                pltpu.make_async_copy(table.at[c].at[cur_s.at[ch]], rows[bb],
                                      gsem[bb]).wait()
                pltpu.async_copy(rows[bb], acc_sh.at[cur_d.at[ch]], ssem[bb],
                                 add=True)
                if bb == 2:
                    @pl.when((ch == 2) & (b + 1 < nblk))
                    def _():
                        pltpu.async_copy(srcb.at[s,
                                                 pl.ds((b + 1) * BCH, BCH)],
                                         nxt_s, isem)
                        pltpu.async_copy(dstg.at[s, pl.ds((b + 1) * BCH, BCH)],
                                         nxt_d, isem)

                @pl.when(g >= 2)
                def _():
                    pltpu.make_async_copy(rows[nb], acc_sh.at[cur_d.at[ch]],
                                          ssem[nb]).wait()

                @pl.when(ch + 2 < BCH)
                def _():
                    pltpu.async_copy(table.at[c].at[cur_s.at[ch + 2]], rows[nb],
                                     gsem[nb])

                @pl.when((ch + 2 >= BCH) & (b + 1 < nblk))
                def _():
                    @pl.when(ch == BCH - 2)
                    def _():
                        pltpu.make_async_copy(
                            srcb.at[s, pl.ds(0, BCH)], nxt_s, isem).wait()
                        pltpu.make_async_copy(
                            dstg.at[s, pl.ds(0, BCH)], nxt_d, isem).wait()

                    pltpu.async_copy(table.at[c].at[nxt_s.at[ch + 2 - BCH]],
                                     rows[nb], gsem[nb])

            return carry

        lax.fori_loop(0, BCH // 4, quad, 0)

    def pairs(pp, carry):
        process(2 * pp, sidxA, didxA, sidxB, didxB)
        process(2 * pp + 1, sidxB, didxB, sidxA, didxA)
        return carry

    lax.fori_loop(0, nblk // 2, pairs, 0)
    pltpu.make_async_copy(rows2, acc_sh.at[didxA.at[0]], ssem[2]).wait()
    pltpu.make_async_copy(rows3, acc_sh.at[didxA.at[0]], ssem[3]).wait()

    plsc.subcore_barrier()
    pltpu.sync_copy(acc_sh.at[pl.ds(s * RPT, RPT)],
                    out.at[c, pl.ds(s * RPT, RPT)])


def _mm1_body(x_ref, od_ref, w_ref, out_ref):
    od = od_ref[0] + od_ref[1]
    nsrc = jnp.where(od > 0, lax.rsqrt(jnp.maximum(od, 1.0)), 0.0)
    xs = (x_ref[...] * nsrc[:, None]).astype(jnp.bfloat16)
    y = jnp.dot(xs, w_ref[...].astype(jnp.bfloat16),
                preferred_element_type=jnp.float32)
    out_ref[0] = y[:, :FH]
    out_ref[1] = y[:, FH:]


def _mm1(xp, odp, w1p):
    return pl.pallas_call(
        _mm1_body,
        grid=(NBLK,),
        in_specs=[
            pl.BlockSpec((RB, DIN), lambda i: (i, 0)),
            pl.BlockSpec((NC, RB), lambda i: (0, i)),
            pl.BlockSpec((DIN, F), lambda i: (0, 0)),
        ],
        out_specs=pl.BlockSpec((NC, RB, FH), lambda i: (0, i, 0)),
        out_shape=jax.ShapeDtypeStruct((NC, NPAD, FH), jnp.float32),
    )(xp, odp, w1p)


def _mm2_body(agg_ref, od_ref, id_ref, b_ref, w_ref, out_ref):
    od = od_ref[0] + od_ref[1]
    idg = id_ref[0] + id_ref[1]
    nsrc = jnp.where(od > 0, lax.rsqrt(jnp.maximum(od, 1.0)), 0.0)
    ndst = jnp.where(idg > 0, lax.rsqrt(jnp.maximum(idg, 1.0)), 0.0)
    agg = jnp.concatenate([agg_ref[0], agg_ref[1]], axis=1)
    h = jnp.maximum(agg * ndst[:, None] + b_ref[...], 0.0)
    hs = (h * nsrc[:, None]).astype(jnp.bfloat16)
    y = jnp.dot(hs, w_ref[...].astype(jnp.bfloat16),
                preferred_element_type=jnp.float32)
    out_ref[0] = y[:, :FH]
    out_ref[1] = y[:, FH:]


def _mm2(agg, odp, idp, b1p, w2p):
    return pl.pallas_call(
        _mm2_body,
        grid=(NBLK,),
        in_specs=[
            pl.BlockSpec((NC, RB, FH), lambda i: (0, i, 0)),
            pl.BlockSpec((NC, RB), lambda i: (0, i)),
            pl.BlockSpec((NC, RB), lambda i: (0, i)),
            pl.BlockSpec((1, F), lambda i: (0, 0)),
            pl.BlockSpec((F, F), lambda i: (0, 0)),
        ],
        out_specs=pl.BlockSpec((NC, RB, FH), lambda i: (0, i, 0)),
        out_shape=jax.ShapeDtypeStruct((NC, NPAD, FH), jnp.float32),
    )(agg, odp, idp, b1p, w2p)


def _fin_body(agg_ref, id_ref, b_ref, wfc_ref, out_ref):
    i = pl.program_id(0)
    idg = id_ref[0] + id_ref[1]
    ndst = jnp.where(idg > 0, lax.rsqrt(jnp.maximum(idg, 1.0)), 0.0)
    agg = jnp.concatenate([agg_ref[0], agg_ref[1]], axis=1)
    h = jnp.maximum(agg * ndst[:, None] + b_ref[...], 0.0)
    rows = i * RB + lax.broadcasted_iota(jnp.int32, (RB, 1), 0)
    h = jnp.where(rows < N, h, 0.0)
    part = jnp.sum(h * wfc_ref[...], dtype=jnp.float32) / N

    @pl.when(i == 0)
    def _():
        out_ref[0, 0] = 0.0

    out_ref[0, 0] += part


def _fin(agg, idp, b2p, wfcp):
    return pl.pallas_call(
        _fin_body,
        grid=(NBLK,),
        in_specs=[
            pl.BlockSpec((NC, RB, FH), lambda i: (0, i, 0)),
            pl.BlockSpec((NC, RB), lambda i: (0, i)),
            pl.BlockSpec((1, F), lambda i: (0, 0)),
            pl.BlockSpec((1, F), lambda i: (0, 0)),
        ],
        out_specs=pl.BlockSpec((1, 1), lambda i: (0, 0),
                               memory_space=pltpu.SMEM),
        out_shape=jax.ShapeDtypeStruct((1, 1), jnp.float32),
    )(agg, idp, b2p, wfcp)


def kernel(x, edge_index, W1, b1, W2, b2, W_fc, b_fc):
    src = edge_index[0].astype(jnp.int32)
    dst = edge_index[1].astype(jnp.int32)
    npad_e = EPAD - E
    padr = (jnp.arange(npad_e, dtype=jnp.int32) % (NPAD - N)) + N
    srcb = jnp.concatenate([src, padr]).reshape(NS, EC, CHUNK)
    dstb = jnp.concatenate([dst, padr]).reshape(NS, EC, CHUNK)

    xp = jnp.pad(x, ((0, NPAD - N), (0, 0)))
    w1p = jnp.pad(W1, ((0, 0), (0, F - H)))
    w2p = jnp.pad(W2, ((0, F - H), (0, F - H)))
    b1p = jnp.pad(b1, (0, F - H)).reshape(1, F)
    b2p = jnp.pad(b2, (0, F - H)).reshape(1, F)
    wfcp = jnp.pad(W_fc, ((0, 0), (0, F - H)))

    ones_h = jnp.ones((CHUNK,), jnp.float32)
    zeros_1d = jnp.zeros((RPT,), jnp.float32)
    zeros_2d = jnp.zeros((RPT, FH), jnp.float32)

    deg_call, seg_call = _sc_calls()
    deg = deg_call(srcb, dstb, ones_h, zeros_1d)
    odp = deg[:, 0, :]
    idp = deg[:, 1, :]

    t1 = _mm1(xp, odp, w1p)
    agg1 = seg_call(srcb, dstb, t1, zeros_2d)
    t2 = _mm2(agg1, odp, idp, b1p, w2p)
    agg2 = seg_call(srcb, dstb, t2, zeros_2d)
    out = _fin(agg2, idp, b2p, wfcp)
    return out.reshape(1) + b_fc


# speedup vs baseline: 9.8827x; 1.0272x over previous
import functools

import jax
import jax.numpy as jnp
from jax import lax
from jax.experimental import pallas as pl
from jax.experimental.pallas import tpu as pltpu
from jax.experimental.pallas import tpu_sc as plsc

N, E, DIN, H = 10000, 320000, 128, 246

NC, NS, LANES = 2, 16, 16
NPAD = 10240
F = 256
FH = F // NC
CHUNK = 80
EC = 256
EPAD = NS * EC * CHUNK
ECH = EC // NC
BCH = 16
RPT = NPAD // NS
RB = 1280
NBLK = NPAD // RB


def _deg_body(srcb, dstg, ones_h, zeros_h, out, sidx, didx, ones_v,
              od_sh, id_sh, o0, o1, o2, o3, i0, i1, i2, i3):
    c = lax.axis_index("c")
    s = lax.axis_index("s")
    osem = (o0, o1, o2, o3)
    isem = (i0, i1, i2, i3)
    pltpu.sync_copy(zeros_h, od_sh.at[pl.ds(s * RPT, RPT)])
    pltpu.sync_copy(zeros_h, id_sh.at[pl.ds(s * RPT, RPT)])
    pltpu.sync_copy(ones_h, ones_v)
    pltpu.sync_copy(srcb.at[s, pl.ds(c * ECH, ECH)], sidx)
    pltpu.sync_copy(dstg.at[s, pl.ds(c * ECH, ECH)], didx)
    plsc.subcore_barrier()

    def quad(q, carry):
        for bb in range(4):
            j = 4 * q + bb

            @pl.when(j >= 4)
            def _():
                pltpu.make_async_copy(ones_v, od_sh.at[sidx.at[j]],
                                      osem[bb]).wait()
                pltpu.make_async_copy(ones_v, id_sh.at[didx.at[j]],
                                      isem[bb]).wait()

            pltpu.async_copy(ones_v, od_sh.at[sidx.at[j]], osem[bb], add=True)
            pltpu.async_copy(ones_v, id_sh.at[didx.at[j]], isem[bb], add=True)
        return carry

    lax.fori_loop(0, ECH // 4, quad, 0)
    for bb in range(4):
        pltpu.make_async_copy(ones_v, od_sh.at[sidx.at[0]], osem[bb]).wait()
        pltpu.make_async_copy(ones_v, id_sh.at[didx.at[0]], isem[bb]).wait()
    plsc.subcore_barrier()
    pltpu.sync_copy(od_sh.at[pl.ds(s * RPT, RPT)],
                    out.at[c, 0, pl.ds(s * RPT, RPT)])
    pltpu.sync_copy(id_sh.at[pl.ds(s * RPT, RPT)],
                    out.at[c, 1, pl.ds(s * RPT, RPT)])


@functools.lru_cache(maxsize=None)
def _sc_calls():
    mesh = plsc.VectorSubcoreMesh(core_axis_name="c", subcore_axis_name="s",
                                  num_cores=NC, num_subcores=NS)
    deg_call = pl.kernel(
        _deg_body,
        out_type=jax.ShapeDtypeStruct((NC, 2, NPAD), jnp.float32),
        mesh=mesh,
        scratch_types=[
            pltpu.VMEM((ECH, CHUNK), jnp.int32),
            pltpu.VMEM((ECH, CHUNK), jnp.int32),
            pltpu.VMEM((CHUNK,), jnp.float32),
            pltpu.VMEM_SHARED((NPAD,), jnp.float32),
            pltpu.VMEM_SHARED((NPAD,), jnp.float32),
            pltpu.SemaphoreType.DMA,
            pltpu.SemaphoreType.DMA,
            pltpu.SemaphoreType.DMA,
            pltpu.SemaphoreType.DMA,
            pltpu.SemaphoreType.DMA,
            pltpu.SemaphoreType.DMA,
            pltpu.SemaphoreType.DMA,
            pltpu.SemaphoreType.DMA,
        ],
    )
    seg_call = pl.kernel(
        _seg_body,
        out_type=jax.ShapeDtypeStruct((NC, NPAD, FH), jnp.float32),
        mesh=mesh,
        scratch_types=[
            pltpu.VMEM((BCH, CHUNK), jnp.int32),
            pltpu.VMEM((BCH, CHUNK), jnp.int32),
            pltpu.VMEM((BCH, CHUNK), jnp.int32),
            pltpu.VMEM((BCH, CHUNK), jnp.int32),
            pltpu.VMEM((CHUNK, FH), jnp.float32),
            pltpu.VMEM((CHUNK, FH), jnp.float32),
            pltpu.VMEM((CHUNK, FH), jnp.float32),
            pltpu.VMEM((CHUNK, FH), jnp.float32),
            pltpu.VMEM_SHARED((NPAD, FH), jnp.float32),
            pltpu.SemaphoreType.DMA,
            pltpu.SemaphoreType.DMA,
            pltpu.SemaphoreType.DMA,
            pltpu.SemaphoreType.DMA,
            pltpu.SemaphoreType.DMA,
            pltpu.SemaphoreType.DMA,
            pltpu.SemaphoreType.DMA,
            pltpu.SemaphoreType.DMA,
            pltpu.SemaphoreType.DMA,
        ],
    )
    return deg_call, seg_call


def _seg_body(srcb, dstg, table, zeros_h, out, sidxA, didxA, sidxB, didxB,
              rows0, rows1, rows2, rows3, acc_sh,
              g0, g1, g2, g3, s0, s1, s2, s3, isem):
    c = lax.axis_index("c")
    s = lax.axis_index("s")
    rows = (rows0, rows1, rows2, rows3)
    gsem = (g0, g1, g2, g3)
    ssem = (s0, s1, s2, s3)
    nblk = EC // BCH
    pltpu.sync_copy(srcb.at[s, pl.ds(0, BCH)], sidxA)
    pltpu.sync_copy(dstg.at[s, pl.ds(0, BCH)], didxA)
    pltpu.async_copy(table.at[c].at[sidxA.at[0]], rows0, gsem[0])
    pltpu.async_copy(table.at[c].at[sidxA.at[1]], rows1, gsem[1])
    pltpu.sync_copy(zeros_h, acc_sh.at[pl.ds(s * RPT, RPT)])
    plsc.subcore_barrier()

    def process(b, cur_s, cur_d, nxt_s, nxt_d):
        def quad(q, carry):
            for bb in range(4):
                ch = 4 * q + bb
                g = b * BCH + ch
                nb = (bb + 2) % 4
                pltpu.make_async_copy(table.at[c].at[cur_s.at[ch]], rows[bb],
                                      gsem[bb]).wait()
                pltpu.async_copy(rows[bb], acc_sh.at[cur_d.at[ch]], ssem[bb],
                                 add=True)
                if bb == 2:
                    @pl.when((ch == 2) & (b + 1 < nblk))
                    def _():
                        pltpu.async_copy(srcb.at[s,
                                                 pl.ds((b + 1) * BCH, BCH)],
                                         nxt_s, isem)
                        pltpu.async_copy(dstg.at[s, pl.ds((b + 1) * BCH, BCH)],
                                         nxt_d, isem)

                @pl.when(g >= 2)
                def _():
                    pltpu.make_async_copy(rows[nb], acc_sh.at[cur_d.at[ch]],
                                          ssem[nb]).wait()

                @pl.when(ch + 2 < BCH)
                def _():
                    pltpu.async_copy(table.at[c].at[cur_s.at[ch + 2]], rows[nb],
                                     gsem[nb])

                @pl.when((ch + 2 >= BCH) & (b + 1 < nblk))
                def _():
                    @pl.when(ch == BCH - 2)
                    def _():
                        pltpu.make_async_copy(
                            srcb.at[s, pl.ds(0, BCH)], nxt_s, isem).wait()
                        pltpu.make_async_copy(
                            dstg.at[s, pl.ds(0, BCH)], nxt_d, isem).wait()

                    pltpu.async_copy(table.at[c].at[nxt_s.at[ch + 2 - BCH]],
                                     rows[nb], gsem[nb])

            return carry

        lax.fori_loop(0, BCH // 4, quad, 0)

    def pairs(pp, carry):
        process(2 * pp, sidxA, didxA, sidxB, didxB)
        process(2 * pp + 1, sidxB, didxB, sidxA, didxA)
        return carry

    lax.fori_loop(0, nblk // 2, pairs, 0)
    pltpu.make_async_copy(rows2, acc_sh.at[didxA.at[0]], ssem[2]).wait()
    pltpu.make_async_copy(rows3, acc_sh.at[didxA.at[0]], ssem[3]).wait()

    plsc.subcore_barrier()
    pltpu.sync_copy(acc_sh.at[pl.ds(s * RPT, RPT)],
                    out.at[c, pl.ds(s * RPT, RPT)])


def _mm1_body(x_ref, od_ref, w_ref, out_ref):
    od = od_ref[0] + od_ref[1]
    nsrc = jnp.where(od > 0, lax.rsqrt(jnp.maximum(od, 1.0)), 0.0)
    xs = (x_ref[...] * nsrc[:, None]).astype(jnp.bfloat16)
    y = jnp.dot(xs, w_ref[...].astype(jnp.bfloat16),
                preferred_element_type=jnp.float32)
    out_ref[0] = y[:, :FH]
    out_ref[1] = y[:, FH:]


def _mm1(xp, odp, w1p):
    return pl.pallas_call(
        _mm1_body,
        grid=(NBLK,),
        in_specs=[
            pl.BlockSpec((RB, DIN), lambda i: (i, 0)),
            pl.BlockSpec((NC, RB), lambda i: (0, i)),
            pl.BlockSpec((DIN, F), lambda i: (0, 0)),
        ],
        out_specs=pl.BlockSpec((NC, RB, FH), lambda i: (0, i, 0)),
        out_shape=jax.ShapeDtypeStruct((NC, NPAD, FH), jnp.float32),
    )(xp, odp, w1p)


def _mm2_body(agg_ref, od_ref, id_ref, b_ref, w_ref, out_ref):
    od = od_ref[0] + od_ref[1]
    idg = id_ref[0] + id_ref[1]
    nsrc = jnp.where(od > 0, lax.rsqrt(jnp.maximum(od, 1.0)), 0.0)
    ndst = jnp.where(idg > 0, lax.rsqrt(jnp.maximum(idg, 1.0)), 0.0)
    agg = jnp.concatenate([agg_ref[0], agg_ref[1]], axis=1)
    h = jnp.maximum(agg * ndst[:, None] + b_ref[...], 0.0)
    hs = (h * nsrc[:, None]).astype(jnp.bfloat16)
    y = jnp.dot(hs, w_ref[...].astype(jnp.bfloat16),
                preferred_element_type=jnp.float32)
    out_ref[0] = y[:, :FH]
    out_ref[1] = y[:, FH:]


def _mm2(agg, odp, idp, b1p, w2p):
    return pl.pallas_call(
        _mm2_body,
        grid=(NBLK,),
        in_specs=[
            pl.BlockSpec((NC, RB, FH), lambda i: (0, i, 0)),
            pl.BlockSpec((NC, RB), lambda i: (0, i)),
            pl.BlockSpec((NC, RB), lambda i: (0, i)),
            pl.BlockSpec((1, F), lambda i: (0, 0)),
            pl.BlockSpec((F, F), lambda i: (0, 0)),
        ],
        out_specs=pl.BlockSpec((NC, RB, FH), lambda i: (0, i, 0)),
        out_shape=jax.ShapeDtypeStruct((NC, NPAD, FH), jnp.float32),
    )(agg, odp, idp, b1p, w2p)


def _fin_body(agg_ref, id_ref, b_ref, wfc_ref, out_ref):
    i = pl.program_id(0)
    idg = id_ref[0] + id_ref[1]
    ndst = jnp.where(idg > 0, lax.rsqrt(jnp.maximum(idg, 1.0)), 0.0)
    agg = jnp.concatenate([agg_ref[0], agg_ref[1]], axis=1)
    h = jnp.maximum(agg * ndst[:, None] + b_ref[...], 0.0)
    rows = i * RB + lax.broadcasted_iota(jnp.int32, (RB, 1), 0)
    h = jnp.where(rows < N, h, 0.0)
    part = jnp.sum(h * wfc_ref[...], dtype=jnp.float32) / N

    @pl.when(i == 0)
    def _():
        out_ref[0, 0] = 0.0

    out_ref[0, 0] += part


def _fin(agg, idp, b2p, wfcp):
    return pl.pallas_call(
        _fin_body,
        grid=(NBLK,),
        in_specs=[
            pl.BlockSpec((NC, RB, FH), lambda i: (0, i, 0)),
            pl.BlockSpec((NC, RB), lambda i: (0, i)),
            pl.BlockSpec((1, F), lambda i: (0, 0)),
            pl.BlockSpec((1, F), lambda i: (0, 0)),
        ],
        out_specs=pl.BlockSpec((1, 1), lambda i: (0, 0),
                               memory_space=pltpu.SMEM),
        out_shape=jax.ShapeDtypeStruct((1, 1), jnp.float32),
    )(agg, idp, b2p, wfcp)


def kernel(x, edge_index, W1, b1, W2, b2, W_fc, b_fc):
    src = edge_index[0].astype(jnp.int32)
    dst = edge_index[1].astype(jnp.int32)
    npad_e = EPAD - E
    padr = (jnp.arange(npad_e, dtype=jnp.int32) % (NPAD - N)) + N
    srcb = jnp.concatenate([src, padr]).reshape(NS, EC, CHUNK)
    dstb = jnp.concatenate([dst, padr]).reshape(NS, EC, CHUNK)

    xp = jnp.pad(x, ((0, NPAD - N), (0, 0)))
    w1p = jnp.pad(W1, ((0, 0), (0, F - H)))
    w2p = jnp.pad(W2, ((0, F - H), (0, F - H)))
    b1p = jnp.pad(b1, (0, F - H)).reshape(1, F)
    b2p = jnp.pad(b2, (0, F - H)).reshape(1, F)
    wfcp = jnp.pad(W_fc, ((0, 0), (0, F - H)))

    ones_h = jnp.ones((CHUNK,), jnp.float32)
    zeros_1d = jnp.zeros((RPT,), jnp.float32)
    zeros_2d = jnp.zeros((RPT, FH), jnp.float32)

    deg_call, seg_call = _sc_calls()
    deg = deg_call(srcb, dstb, ones_h, zeros_1d)
    odp = deg[:, 0, :]
    idp = deg[:, 1, :]

    t1 = _mm1(xp, odp, w1p)
    agg1 = seg_call(srcb, dstb, t1, zeros_2d)
    t2 = _mm2(agg1, odp, idp, b1p, w2p)
    agg2 = seg_call(srcb, dstb, t2, zeros_2d)
    out = _fin(agg2, idp, b2p, wfcp)
    return out.reshape(1) + b_fc
